# baseline (speedup 1.0000x reference)
import functools

import jax
import jax.numpy as jnp
from jax import lax
from jax.experimental import pallas as pl
from jax.experimental.pallas import tpu as pltpu

N_CLUSTERS = 32
DELTA_DIST = 1.5
W_VAR, W_DIST, W_REG, W_SMOOTH, W_SEED = 3.0, 1.0, 0.001, 5.0, 5.0

_LOG_LO = float(jnp.log(jnp.float32(1e-12)))
_LOG_HI = float(jnp.log1p(jnp.float32(-1e-12)))


def _sum11(a):
    return jnp.sum(jnp.sum(a, axis=1, keepdims=True), axis=0, keepdims=True)


def _loss_kernel(x_ref, st_ref, lab_ref,
                 loss_ref, varl_ref, distl_ref, regl_ref, seedl_ref, smoothl_ref,
                 acc_sx, acc_cnt, acc_se, acc_se2,
                 cm_s, c2_s, i2v_s,
                 a_l1mp, a_own, a_seed,
                 *, nblocks, n_total):
    phase = pl.program_id(0)
    j = pl.program_id(1)
    C = N_CLUSTERS
    f32 = jnp.float32

    labels = lab_ref[...]
    B = labels.shape[1]
    cid = lax.broadcasted_iota(jnp.int32, (C, B), 0)
    onehot = (cid == labels)

    @pl.when(jnp.logical_and(phase == 0, j == 0))
    def _init():
        acc_sx[...] = jnp.zeros_like(acc_sx)
        acc_cnt[...] = jnp.zeros_like(acc_cnt)
        acc_se[...] = jnp.zeros_like(acc_se)
        acc_se2[...] = jnp.zeros_like(acc_se2)
        a_l1mp[...] = jnp.zeros_like(a_l1mp)
        a_own[...] = jnp.zeros_like(a_own)
        a_seed[...] = jnp.zeros_like(a_seed)

    @pl.when(phase == 0)
    def _stats():
        x = x_ref[...]
        oh = onehot.astype(f32)
        e = jnp.exp(st_ref[1:2, :])
        acc_sx[...] += lax.dot_general(
            oh, x, (((1,), (0,)), ((), ())), preferred_element_type=f32)
        acc_cnt[...] += jnp.sum(oh, axis=1, keepdims=True)
        acc_se[...] += jnp.sum(jnp.where(onehot, e, 0.0), axis=1, keepdims=True)
        acc_se2[...] += jnp.sum(jnp.where(onehot, e * e, 0.0), axis=1, keepdims=True)

    @pl.when(jnp.logical_and(phase == 1, j == 0))
    def _finalize_stats():
        cm = acc_sx[...] / acc_cnt[...]
        cm_s[...] = cm
        c2_s[...] = jnp.sum(cm * cm, axis=1, keepdims=True)
        var_c = acc_se[...] / acc_cnt[...]
        i2v_s[...] = 0.5 / var_c

    @pl.when(phase == 1)
    def _dense():
        x = x_ref[...]
        cm = cm_s[...]
        xc = lax.dot_general(
            cm, x, (((1,), (1,)), ((), ())), preferred_element_type=f32)
        xsq = x * x
        ones_d = jnp.ones((1, x.shape[1]), dtype=f32)
        x2 = lax.dot_general(
            ones_d, xsq, (((1,), (1,)), ((), ())), preferred_element_type=f32)
        d2 = jnp.maximum(x2 - 2.0 * xc + c2_s[...], 0.0)
        t = d2 * i2v_s[...]
        prob = jnp.exp(-t)
        p = jnp.clip(prob, 1e-12, 1.0 - 1e-12)
        logp = jnp.clip(-t, _LOG_LO, _LOG_HI)
        log1mp = jnp.log1p(-p)
        a_l1mp[...] += _sum11(log1mp)
        a_own[...] += _sum11(jnp.where(onehot, logp - log1mp, 0.0))
        prob_own = jnp.sum(jnp.where(onehot, prob, 0.0), axis=0, keepdims=True)
        s0 = st_ref[0:1, :]
        dsq = prob_own - s0
        a_seed[...] += _sum11(dsq * dsq)

    @pl.when(jnp.logical_and(phase == 1, j == nblocks - 1))
    def _combine():
        n = f32(n_total)
        cm = cm_s[...]
        c2col = c2_s[...]
        cmsq = cm * cm
        ones_d = jnp.ones((1, cm.shape[1]), dtype=f32)
        c2row = lax.dot_general(
            ones_d, cmsq, (((1,), (1,)), ((), ())), preferred_element_type=f32)
        gram = lax.dot_general(
            cm, cm, (((1,), (1,)), ((), ())), preferred_element_type=f32)
        r = lax.broadcasted_iota(jnp.int32, (C, C), 0)
        cc = lax.broadcasted_iota(jnp.int32, (C, C), 1)
        eye = (r == cc).astype(f32)
        dmat = jnp.sqrt(jnp.maximum(c2col + c2row - 2.0 * gram, 0.0) + eye)
        hinge = jnp.maximum(2.0 * DELTA_DIST - dmat, 0.0)
        dist_loss = _sum11(hinge * hinge * (1.0 - eye)) / f32((C - 1) * C)
        reg_loss = _sum11(jnp.sqrt(c2col)) / f32(C)
        var_c = acc_se[...] / acc_cnt[...]
        smooth_c = acc_se2[...] / acc_cnt[...] - var_c * var_c
        smoothness_loss = _sum11(smooth_c) / f32(C)
        var_loss = -(a_l1mp[...] + a_own[...]) / (n * f32(C))
        seed_loss = a_seed[...] / n
        loss = (W_VAR * var_loss + W_DIST * dist_loss + W_REG * reg_loss
                + W_SMOOTH * smoothness_loss + W_SEED * seed_loss)
        loss_ref[...] = loss
        varl_ref[...] = W_VAR * var_loss
        distl_ref[...] = W_DIST * dist_loss
        regl_ref[...] = W_REG * reg_loss
        seedl_ref[...] = W_SEED * seed_loss
        smoothl_ref[...] = W_SMOOTH * smoothness_loss


@jax.jit
def kernel(features, seediness, group_labels):
    n, d = features.shape
    blk = 8192
    nblocks = n // blk
    st = seediness.T
    lab = group_labels.reshape(1, n).astype(jnp.int32)

    scalar = jax.ShapeDtypeStruct((1, 1), jnp.float32)
    out = pl.pallas_call(
        functools.partial(_loss_kernel, nblocks=nblocks, n_total=n),
        grid=(2, nblocks),
        in_specs=[
            pl.BlockSpec((blk, d), lambda p, j: (j, 0)),
            pl.BlockSpec((2, blk), lambda p, j: (0, j)),
            pl.BlockSpec((1, blk), lambda p, j: (0, j)),
        ],
        out_specs=[pl.BlockSpec((1, 1), lambda p, j: (0, 0))] * 6,
        out_shape=[scalar] * 6,
        scratch_shapes=[
            pltpu.VMEM((N_CLUSTERS, d), jnp.float32),
            pltpu.VMEM((N_CLUSTERS, 1), jnp.float32),
            pltpu.VMEM((N_CLUSTERS, 1), jnp.float32),
            pltpu.VMEM((N_CLUSTERS, 1), jnp.float32),
            pltpu.VMEM((N_CLUSTERS, d), jnp.float32),
            pltpu.VMEM((N_CLUSTERS, 1), jnp.float32),
            pltpu.VMEM((N_CLUSTERS, 1), jnp.float32),
            pltpu.VMEM((1, 1), jnp.float32),
            pltpu.VMEM((1, 1), jnp.float32),
            pltpu.VMEM((1, 1), jnp.float32),
        ],
    )(features, st, lab)
    return tuple(o.reshape(()) for o in out)


# speedup vs baseline: 2.4918x; 2.4918x over previous
import functools
import math

import jax
import jax.numpy as jnp
from jax import lax
from jax.experimental import pallas as pl
from jax.experimental.pallas import tpu as pltpu

N_CLUSTERS = 32
DELTA_DIST = 1.5
W_VAR, W_DIST, W_REG, W_SMOOTH, W_SEED = 3.0, 1.0, 0.001, 5.0, 5.0

_LOG_LO = math.log(1e-12)
_LOG_HI = math.log1p(-1e-12)


def _sum11(a):
    return jnp.sum(jnp.sum(a, axis=1, keepdims=True), axis=0, keepdims=True)


def _loss_kernel(x_ref, st_ref, lab_ref,
                 loss_ref, varl_ref, distl_ref, regl_ref, seedl_ref, smoothl_ref,
                 acc_all, cm_s, c2_s, i2v_s,
                 a_l1mp, a_own, a_seed,
                 *, nblocks, n_total):
    phase = pl.program_id(0)
    j = pl.program_id(1)
    C = N_CLUSTERS
    f32 = jnp.float32

    labels = lab_ref[...]
    B = labels.shape[1]
    cid = lax.broadcasted_iota(jnp.int32, (C, B), 0)
    oh = (cid == labels).astype(f32)

    @pl.when(jnp.logical_and(phase == 0, j == 0))
    def _init():
        acc_all[...] = jnp.zeros_like(acc_all)
        a_l1mp[...] = jnp.zeros_like(a_l1mp)
        a_own[...] = jnp.zeros_like(a_own)
        a_seed[...] = jnp.zeros_like(a_seed)

    @pl.when(phase == 0)
    def _stats():
        x = x_ref[...]
        e = jnp.exp(st_ref[1:2, :])
        ones_b = jnp.ones((1, B), dtype=f32)
        aug = jnp.concatenate([x, ones_b, e, e * e], axis=0)
        acc_all[...] += lax.dot_general(
            oh, aug, (((1,), (1,)), ((), ())), preferred_element_type=f32)

    @pl.when(jnp.logical_and(phase == 1, j == 0))
    def _finalize_stats():
        d = x_ref.shape[0]
        cnt = acc_all[:, d:d + 1]
        cm = acc_all[:, :d] / cnt
        cm_s[...] = cm
        c2_s[...] = jnp.sum(cm * cm, axis=1, keepdims=True)
        var_c = acc_all[:, d + 1:d + 2] / cnt
        i2v_s[...] = 0.5 / var_c

    @pl.when(phase == 1)
    def _dense():
        x = x_ref[...]
        cm = cm_s[...]
        xc = lax.dot_general(
            cm, x, (((1,), (0,)), ((), ())), preferred_element_type=f32)
        xsq = x * x
        ones_d = jnp.ones((1, x.shape[0]), dtype=f32)
        x2 = lax.dot_general(
            ones_d, xsq, (((1,), (0,)), ((), ())), preferred_element_type=f32)
        d2 = jnp.maximum(x2 - 2.0 * xc + c2_s[...], 0.0)
        t = d2 * i2v_s[...]
        prob = jnp.exp(-t)
        p = jnp.clip(prob, 1e-12, 1.0 - 1e-12)
        logp = jnp.clip(-t, _LOG_LO, _LOG_HI)
        log1mp = jnp.log1p(-p)
        a_l1mp[...] += _sum11(log1mp)
        a_own[...] += _sum11(oh * (logp - log1mp))
        prob_own = jnp.sum(oh * prob, axis=0, keepdims=True)
        dsq = prob_own - st_ref[0:1, :]
        a_seed[...] += _sum11(dsq * dsq)

    @pl.when(jnp.logical_and(phase == 1, j == nblocks - 1))
    def _combine():
        n = f32(n_total)
        d = x_ref.shape[0]
        cnt = acc_all[:, d:d + 1]
        cm = cm_s[...]
        c2col = c2_s[...]
        cmsq = cm * cm
        ones_d = jnp.ones((1, d), dtype=f32)
        c2row = lax.dot_general(
            ones_d, cmsq, (((1,), (1,)), ((), ())), preferred_element_type=f32)
        gram = lax.dot_general(
            cm, cm, (((1,), (1,)), ((), ())), preferred_element_type=f32)
        r = lax.broadcasted_iota(jnp.int32, (C, C), 0)
        cc = lax.broadcasted_iota(jnp.int32, (C, C), 1)
        eye = (r == cc).astype(f32)
        dmat = jnp.sqrt(jnp.maximum(c2col + c2row - 2.0 * gram, 0.0) + eye)
        hinge = jnp.maximum(2.0 * DELTA_DIST - dmat, 0.0)
        dist_loss = _sum11(hinge * hinge * (1.0 - eye)) / f32((C - 1) * C)
        reg_loss = _sum11(jnp.sqrt(c2col)) / f32(C)
        var_c = acc_all[:, d + 1:d + 2] / cnt
        smooth_c = acc_all[:, d + 2:d + 3] / cnt - var_c * var_c
        smoothness_loss = _sum11(smooth_c) / f32(C)
        var_loss = -(a_l1mp[...] + a_own[...]) / (n * f32(C))
        seed_loss = a_seed[...] / n
        loss = (W_VAR * var_loss + W_DIST * dist_loss + W_REG * reg_loss
                + W_SMOOTH * smoothness_loss + W_SEED * seed_loss)
        loss_ref[...] = loss
        varl_ref[...] = W_VAR * var_loss
        distl_ref[...] = W_DIST * dist_loss
        regl_ref[...] = W_REG * reg_loss
        seedl_ref[...] = W_SEED * seed_loss
        smoothl_ref[...] = W_SMOOTH * smoothness_loss


@jax.jit
def kernel(features, seediness, group_labels):
    n, d = features.shape
    blk = 8192
    nblocks = n // blk
    xt = features.T
    st = seediness.T
    lab = group_labels.reshape(1, n).astype(jnp.int32)

    scalar = jax.ShapeDtypeStruct((1, 1), jnp.float32)
    out = pl.pallas_call(
        functools.partial(_loss_kernel, nblocks=nblocks, n_total=n),
        grid=(2, nblocks),
        in_specs=[
            pl.BlockSpec((d, blk), lambda p, j: (0, j)),
            pl.BlockSpec((2, blk), lambda p, j: (0, j)),
            pl.BlockSpec((1, blk), lambda p, j: (0, j)),
        ],
        out_specs=[pl.BlockSpec((1, 1), lambda p, j: (0, 0))] * 6,
        out_shape=[scalar] * 6,
        scratch_shapes=[
            pltpu.VMEM((N_CLUSTERS, 19), jnp.float32),
            pltpu.VMEM((N_CLUSTERS, 16), jnp.float32),
            pltpu.VMEM((N_CLUSTERS, 1), jnp.float32),
            pltpu.VMEM((N_CLUSTERS, 1), jnp.float32),
            pltpu.VMEM((1, 1), jnp.float32),
            pltpu.VMEM((1, 1), jnp.float32),
            pltpu.VMEM((1, 1), jnp.float32),
        ],
    )(xt, st, lab)
    return tuple(o.reshape(()) for o in out)


# speedup vs baseline: 2.6821x; 1.0764x over previous
import functools
import math

import jax
import jax.numpy as jnp
from jax import lax
from jax.experimental import pallas as pl
from jax.experimental.pallas import tpu as pltpu

N_CLUSTERS = 32
DELTA_DIST = 1.5
W_VAR, W_DIST, W_REG, W_SMOOTH, W_SEED = 3.0, 1.0, 0.001, 5.0, 5.0

_LOG_LO = math.log(1e-12)
_PMAX = 1.0 - 1e-12


def _sum11(a):
    return jnp.sum(jnp.sum(a, axis=0, keepdims=True), axis=1, keepdims=True)


def _loss_kernel(x_ref, st_ref, lab_ref,
                 loss_ref, varl_ref, distl_ref, regl_ref, seedl_ref, smoothl_ref,
                 acc_all, cm_s, c2_s, i2v_s,
                 row_bce, row_seed,
                 *, nblocks, n_total):
    phase = pl.program_id(0)
    j = pl.program_id(1)
    C = N_CLUSTERS
    f32 = jnp.float32

    labels = lab_ref[...]
    B = labels.shape[1]
    cid = lax.broadcasted_iota(jnp.int32, (C, B), 0)
    onehot = (cid == labels)

    @pl.when(jnp.logical_and(phase == 0, j == 0))
    def _init():
        acc_all[...] = jnp.zeros_like(acc_all)

    @pl.when(phase == 0)
    def _stats():
        x = x_ref[...]
        e = jnp.exp(st_ref[1:2, :])
        ones_b = jnp.ones((1, B), dtype=f32)
        aug = jnp.concatenate([x, ones_b, e, e * e], axis=0)
        oh = onehot.astype(f32)
        acc_all[...] += lax.dot_general(
            oh, aug, (((1,), (1,)), ((), ())), preferred_element_type=f32)

    @pl.when(jnp.logical_and(phase == 1, j == 0))
    def _finalize_stats():
        d = x_ref.shape[0]
        cnt = acc_all[:, d:d + 1]
        cm = acc_all[:, :d] / cnt
        cm_s[...] = cm
        c2_s[...] = jnp.sum(cm * cm, axis=1, keepdims=True)
        var_c = acc_all[:, d + 1:d + 2] / cnt
        i2v_s[...] = 0.5 / var_c

    @pl.when(phase == 1)
    def _dense():
        x = x_ref[...]
        cm = cm_s[...]
        xc = lax.dot_general(
            cm, x, (((1,), (0,)), ((), ())), preferred_element_type=f32)
        xsq = x * x
        ones_d = jnp.ones((1, x.shape[0]), dtype=f32)
        x2 = lax.dot_general(
            ones_d, xsq, (((1,), (0,)), ((), ())), preferred_element_type=f32)
        d2 = jnp.maximum(x2 - 2.0 * xc + c2_s[...], 0.0)
        t = d2 * i2v_s[...]
        prob = jnp.exp(-t)
        p = jnp.minimum(prob, _PMAX)
        log1mp = jnp.log1p(-p)
        fold8 = (log1mp[0:8, :] + log1mp[8:16, :]
                 + log1mp[16:24, :] + log1mp[24:32, :])
        prob_own = jnp.sum(jnp.where(onehot, prob, 0.0), axis=0, keepdims=True)
        lp_own = jnp.maximum(jnp.log(prob_own), _LOG_LO)
        l1mp_own = jnp.log1p(-jnp.minimum(prob_own, _PMAX))
        keep = j != 0
        row_bce[...] = jnp.where(keep, row_bce[...], 0.0) + fold8
        row_bce[0:1, :] += lp_own - l1mp_own
        dsq = prob_own - st_ref[0:1, :]
        row_seed[...] = jnp.where(keep, row_seed[...], 0.0) + dsq * dsq

    @pl.when(jnp.logical_and(phase == 1, j == nblocks - 1))
    def _combine():
        n = f32(n_total)
        d = x_ref.shape[0]
        cnt = acc_all[:, d:d + 1]
        cm = cm_s[...]
        c2col = c2_s[...]
        cmsq = cm * cm
        ones_d = jnp.ones((1, d), dtype=f32)
        c2row = lax.dot_general(
            ones_d, cmsq, (((1,), (1,)), ((), ())), preferred_element_type=f32)
        gram = lax.dot_general(
            cm, cm, (((1,), (1,)), ((), ())), preferred_element_type=f32)
        r = lax.broadcasted_iota(jnp.int32, (C, C), 0)
        cc = lax.broadcasted_iota(jnp.int32, (C, C), 1)
        eye = (r == cc).astype(f32)
        dmat = jnp.sqrt(jnp.maximum(c2col + c2row - 2.0 * gram, 0.0) + eye)
        hinge = jnp.maximum(2.0 * DELTA_DIST - dmat, 0.0)
        dist_loss = _sum11(hinge * hinge * (1.0 - eye)) / f32((C - 1) * C)
        reg_loss = _sum11(jnp.sqrt(c2col)) / f32(C)
        var_c = acc_all[:, d + 1:d + 2] / cnt
        smooth_c = acc_all[:, d + 2:d + 3] / cnt - var_c * var_c
        smoothness_loss = _sum11(smooth_c) / f32(C)
        var_loss = -_sum11(row_bce[...]) / (n * f32(C))
        seed_loss = _sum11(row_seed[...]) / n
        loss = (W_VAR * var_loss + W_DIST * dist_loss + W_REG * reg_loss
                + W_SMOOTH * smoothness_loss + W_SEED * seed_loss)
        loss_ref[...] = loss
        varl_ref[...] = W_VAR * var_loss
        distl_ref[...] = W_DIST * dist_loss
        regl_ref[...] = W_REG * reg_loss
        seedl_ref[...] = W_SEED * seed_loss
        smoothl_ref[...] = W_SMOOTH * smoothness_loss


@jax.jit
def kernel(features, seediness, group_labels):
    n, d = features.shape
    blk = 8192
    nblocks = n // blk
    xt = features.T
    st = seediness.T
    lab = group_labels.reshape(1, n).astype(jnp.int32)

    scalar = jax.ShapeDtypeStruct((1, 1), jnp.float32)
    out = pl.pallas_call(
        functools.partial(_loss_kernel, nblocks=nblocks, n_total=n),
        grid=(2, nblocks),
        in_specs=[
            pl.BlockSpec((d, blk), lambda p, j: (0, j)),
            pl.BlockSpec((2, blk), lambda p, j: (0, j)),
            pl.BlockSpec((1, blk), lambda p, j: (0, j)),
        ],
        out_specs=[pl.BlockSpec((1, 1), lambda p, j: (0, 0))] * 6,
        out_shape=[scalar] * 6,
        scratch_shapes=[
            pltpu.VMEM((N_CLUSTERS, 19), jnp.float32),
            pltpu.VMEM((N_CLUSTERS, 16), jnp.float32),
            pltpu.VMEM((N_CLUSTERS, 1), jnp.float32),
            pltpu.VMEM((N_CLUSTERS, 1), jnp.float32),
            pltpu.VMEM((8, blk), jnp.float32),
            pltpu.VMEM((1, blk), jnp.float32),
        ],
    )(xt, st, lab)
    return tuple(o.reshape(()) for o in out)


# speedup vs baseline: 3.1016x; 1.1564x over previous
import functools
import math

import jax
import jax.numpy as jnp
from jax import lax
from jax.experimental import pallas as pl
from jax.experimental.pallas import tpu as pltpu

N_CLUSTERS = 32
DELTA_DIST = 1.5
W_VAR, W_DIST, W_REG, W_SMOOTH, W_SEED = 3.0, 1.0, 0.001, 5.0, 5.0

_LOG_LO = math.log(1e-12)
_PMAX = 1.0 - 1e-12


def _sum11(a):
    return jnp.sum(jnp.sum(a, axis=0, keepdims=True), axis=1, keepdims=True)


def _loss_kernel(x_ref, st_ref, lab_ref,
                 loss_ref, varl_ref, distl_ref, regl_ref, seedl_ref, smoothl_ref,
                 acc_all, cm_s, c2_s, i2v_s,
                 row_bce, row_seed,
                 *, nblocks, n_total):
    phase = pl.program_id(0)
    j = pl.program_id(1)
    C = N_CLUSTERS
    f32 = jnp.float32

    labels = lab_ref[...]
    B = labels.shape[1]
    cid = lax.broadcasted_iota(jnp.int32, (C, B), 0)
    onehot = (cid == labels)

    @pl.when(jnp.logical_and(phase == 0, j == 0))
    def _init():
        acc_all[...] = jnp.zeros_like(acc_all)

    @pl.when(phase == 0)
    def _stats():
        x = x_ref[...]
        e = jnp.exp(st_ref[1:2, :])
        ones_b = jnp.ones((1, B), dtype=f32)
        aug = jnp.concatenate([x, ones_b, e, e * e], axis=0)
        oh = onehot.astype(f32)
        acc_all[...] += lax.dot_general(
            oh, aug, (((1,), (1,)), ((), ())), preferred_element_type=f32)

    @pl.when(jnp.logical_and(phase == 1, j == 0))
    def _finalize_stats():
        d = x_ref.shape[0]
        cnt = acc_all[:, d:d + 1]
        cm = acc_all[:, :d] / cnt
        cm_s[...] = cm
        c2_s[...] = jnp.sum(cm * cm, axis=1, keepdims=True)
        var_c = acc_all[:, d + 1:d + 2] / cnt
        i2v_s[...] = 0.5 / var_c

    @pl.when(phase == 1)
    def _dense():
        x = x_ref[...]
        cm = cm_s[...]
        xc = lax.dot_general(
            cm, x, (((1,), (0,)), ((), ())), preferred_element_type=f32)
        xsq = x * x
        ones_d = jnp.ones((1, x.shape[0]), dtype=f32)
        x2 = lax.dot_general(
            ones_d, xsq, (((1,), (0,)), ((), ())), preferred_element_type=f32)
        d2 = jnp.maximum(x2 - 2.0 * xc + c2_s[...], 0.0)
        t = d2 * i2v_s[...]
        prob = jnp.exp(-t)
        p = jnp.minimum(prob, _PMAX)
        log1mp = jnp.log1p(-p)
        fold8 = (log1mp[0:8, :] + log1mp[8:16, :]
                 + log1mp[16:24, :] + log1mp[24:32, :])
        prob_own = jnp.sum(jnp.where(onehot, prob, 0.0), axis=0, keepdims=True)
        lp_own = jnp.maximum(jnp.log(prob_own), _LOG_LO)
        l1mp_own = jnp.log1p(-jnp.minimum(prob_own, _PMAX))
        keep = j != 0
        row_bce[...] = jnp.where(keep, row_bce[...], 0.0) + fold8
        row_bce[0:1, :] += lp_own - l1mp_own
        dsq = prob_own - st_ref[0:1, :]
        row_seed[...] = jnp.where(keep, row_seed[...], 0.0) + dsq * dsq

    @pl.when(jnp.logical_and(phase == 1, j == nblocks - 1))
    def _combine():
        n = f32(n_total)
        d = x_ref.shape[0]
        cnt = acc_all[:, d:d + 1]
        cm = cm_s[...]
        c2col = c2_s[...]
        cmsq = cm * cm
        ones_d = jnp.ones((1, d), dtype=f32)
        c2row = lax.dot_general(
            ones_d, cmsq, (((1,), (1,)), ((), ())), preferred_element_type=f32)
        gram = lax.dot_general(
            cm, cm, (((1,), (1,)), ((), ())), preferred_element_type=f32)
        r = lax.broadcasted_iota(jnp.int32, (C, C), 0)
        cc = lax.broadcasted_iota(jnp.int32, (C, C), 1)
        eye = (r == cc).astype(f32)
        dmat = jnp.sqrt(jnp.maximum(c2col + c2row - 2.0 * gram, 0.0) + eye)
        hinge = jnp.maximum(2.0 * DELTA_DIST - dmat, 0.0)
        dist_loss = _sum11(hinge * hinge * (1.0 - eye)) / f32((C - 1) * C)
        reg_loss = _sum11(jnp.sqrt(c2col)) / f32(C)
        var_c = acc_all[:, d + 1:d + 2] / cnt
        smooth_c = acc_all[:, d + 2:d + 3] / cnt - var_c * var_c
        smoothness_loss = _sum11(smooth_c) / f32(C)
        var_loss = -_sum11(row_bce[...]) / (n * f32(C))
        seed_loss = _sum11(row_seed[...]) / n
        loss = (W_VAR * var_loss + W_DIST * dist_loss + W_REG * reg_loss
                + W_SMOOTH * smoothness_loss + W_SEED * seed_loss)
        loss_ref[...] = loss
        varl_ref[...] = W_VAR * var_loss
        distl_ref[...] = W_DIST * dist_loss
        regl_ref[...] = W_REG * reg_loss
        seedl_ref[...] = W_SEED * seed_loss
        smoothl_ref[...] = W_SMOOTH * smoothness_loss


@jax.jit
def kernel(features, seediness, group_labels):
    n, d = features.shape
    blk = 16384
    nblocks = n // blk
    xt = features.T
    st = seediness.T
    lab = group_labels.reshape(1, n).astype(jnp.int32)

    scalar = jax.ShapeDtypeStruct((1, 1), jnp.float32)
    out = pl.pallas_call(
        functools.partial(_loss_kernel, nblocks=nblocks, n_total=n),
        grid=(2, nblocks),
        in_specs=[
            pl.BlockSpec((d, blk), lambda p, j: (0, j)),
            pl.BlockSpec((2, blk), lambda p, j: (0, j)),
            pl.BlockSpec((1, blk), lambda p, j: (0, j)),
        ],
        out_specs=[pl.BlockSpec((1, 1), lambda p, j: (0, 0))] * 6,
        out_shape=[scalar] * 6,
        scratch_shapes=[
            pltpu.VMEM((N_CLUSTERS, 19), jnp.float32),
            pltpu.VMEM((N_CLUSTERS, 16), jnp.float32),
            pltpu.VMEM((N_CLUSTERS, 1), jnp.float32),
            pltpu.VMEM((N_CLUSTERS, 1), jnp.float32),
            pltpu.VMEM((8, blk), jnp.float32),
            pltpu.VMEM((1, blk), jnp.float32),
        ],
    )(xt, st, lab)
    return tuple(o.reshape(()) for o in out)
